# baseline (speedup 1.0000x reference)
import functools

import jax
import jax.numpy as jnp
from jax import lax
from jax.experimental import pallas as pl
from jax.experimental.pallas import tpu as pltpu
from jax.experimental.pallas import tpu_sc as plsc

SEQ = 200
BATCH = 4096
DIM = 64
VOCAB = 1000000

NC = 2
NS = 16
NW = NC * NS
BPW = BATCH // NW
C1 = 128
C2 = SEQ - C1
XCH = 8
LANES = 16
NJ = DIM // LANES

BLK = 128
NFULL = VOCAB // BLK
TAIL0 = NFULL * BLK
TAILN = VOCAB - TAIL0
TSTEPS = NFULL // NW + 1


def _tr_body(tt_hbm, pad_hbm, blk0, blk1, tb0, tb1, tblk, ttb,
             si0, si1, so0, so1):
    wid = lax.axis_index("s") * NC + lax.axis_index("c")
    lane = lax.iota(jnp.int32, LANES)
    rvecs = [lane + k * LANES for k in range(BLK // LANES)]

    def c0_of(i):
        return (wid + NW * i) * BLK

    def fire_in(i, blk, sem):
        @pl.when(c0_of(i) + BLK <= VOCAB)
        def _():
            pltpu.async_copy(tt_hbm.at[:, pl.ds(c0_of(i), BLK)], blk, sem)

    def wait_in(i, blk, sem):
        @pl.when(c0_of(i) + BLK <= VOCAB)
        def _():
            pltpu.make_async_copy(tt_hbm.at[:, pl.ds(c0_of(i), BLK)],
                                  blk, sem).wait()

    fire_in(0, blk0, si0)
    fire_in(1, blk1, si1)

    def step(ii, carry):
        for ph, (blk, tb, sem_i, sem_o) in enumerate(
                ((blk0, tb0, si0, so0), (blk1, tb1, si1, so1))):
            i = 2 * ii + ph
            c0 = c0_of(i)

            @pl.when(c0 + BLK <= VOCAB)
            def _():
                pltpu.make_async_copy(tt_hbm.at[:, pl.ds(c0, BLK)],
                                      blk, sem_i).wait()
                @pl.when(i >= 2)
                def _():
                    pltpu.make_async_copy(
                        tb, pad_hbm.at[pl.ds(c0_of(i - 2), BLK)],
                        sem_o).wait()

                for j in range(DIM):
                    col = jnp.full((LANES,), j, jnp.int32)
                    for k in range(BLK // LANES):
                        vals = blk[j, pl.ds(k * LANES, LANES)]
                        plsc.store_scatter(tb, [rvecs[k], col], vals)
                pltpu.async_copy(tb, pad_hbm.at[pl.ds(c0, BLK)], sem_o)
                fire_in(i + 2, blk, sem_i)
        return carry

    lax.fori_loop(0, (TSTEPS + 1) // 2, step, 0)
    pltpu.make_async_copy(tb0, pad_hbm.at[pl.ds(0, BLK)], so0).wait()
    pltpu.make_async_copy(tb1, pad_hbm.at[pl.ds(0, BLK)], so1).wait()

    @pl.when(wid == (TAIL0 // BLK) % NW)
    def _():
        pltpu.sync_copy(tt_hbm.at[:, pl.ds(TAIL0, TAILN)], tblk)
        for j in range(DIM):
            col = jnp.full((LANES,), j, jnp.int32)
            for k in range(TAILN // LANES):
                vals = tblk[j, pl.ds(k * LANES, LANES)]
                plsc.store_scatter(ttb, [rvecs[k], col], vals)
        pltpu.sync_copy(ttb, pad_hbm.at[pl.ds(TAIL0, TAILN)])


def _pool_body(x_hbm, pad_hbm, out_hbm, raw_v, idx_v, rows0, rows1, out_v,
               sem0, sem1):
    wid = lax.axis_index("s") * NC + lax.axis_index("c")
    base = wid * BPW

    lane = lax.iota(jnp.int32, LANES)

    def stage_chunk(c, carry):
        s0 = c * XCH
        pltpu.sync_copy(x_hbm.at[pl.ds(s0, XCH), pl.ds(base, BPW)], raw_v)
        for sl in range(XCH):
            col = jnp.full((LANES,), s0 + sl, jnp.int32)

            def tr(k, carry2):
                vals = raw_v[sl, pl.ds(k * LANES, LANES)]
                plsc.store_scatter(
                    idx_v, [lane + k * LANES, col], vals)
                return carry2

            lax.fori_loop(0, BPW // LANES, tr, 0, unroll=4)
        return carry

    lax.fori_loop(0, SEQ // XCH, stage_chunk, 0)

    def fire(b, rows, sem):
        pltpu.async_copy(pad_hbm.at[idx_v.at[b, pl.ds(0, C1)]],
                         rows.at[pl.ds(0, C1)], sem)
        pltpu.async_copy(pad_hbm.at[idx_v.at[b, pl.ds(C1, C2)]],
                         rows.at[pl.ds(C1, C2)], sem)

    def drain(b, rows, sem):
        pltpu.make_async_copy(pad_hbm.at[idx_v.at[b, pl.ds(0, C1)]],
                              rows.at[pl.ds(0, C1)], sem).wait()
        pltpu.make_async_copy(pad_hbm.at[idx_v.at[b, pl.ds(C1, C2)]],
                              rows.at[pl.ds(C1, C2)], sem).wait()

    def reduce_rows(b, rows):
        def red(s, accs):
            return tuple(
                jnp.maximum(a, rows[s, pl.ds(j * LANES, LANES)])
                for j, a in enumerate(accs))
        init = tuple(
            jnp.full((LANES,), -jnp.inf, jnp.float32) for _ in range(NJ))
        accs = lax.fori_loop(0, SEQ, red, init, unroll=8)
        for j in range(NJ):
            out_v[b, pl.ds(j * LANES, LANES)] = accs[j]

    fire(0, rows0, sem0)
    fire(1, rows1, sem1)

    def step(i, carry):
        for ph, (rows, sem) in enumerate(((rows0, sem0), (rows1, sem1))):
            b = 2 * i + ph
            drain(b, rows, sem)
            reduce_rows(b, rows)
            nb = b + 2

            @pl.when(nb < BPW)
            def _():
                fire(nb, rows, sem)
        return carry

    lax.fori_loop(0, BPW // 2, step, 0)

    pltpu.sync_copy(out_v, out_hbm.at[pl.ds(base, BPW)])


def kernel(x, table):
    tt = table.T

    mesh = plsc.VectorSubcoreMesh(
        core_axis_name="c", subcore_axis_name="s",
        num_cores=NC, num_subcores=NS)
    params = pltpu.CompilerParams(
        use_tc_tiling_on_sc=True, needs_layout_passes=False)

    transpose = functools.partial(
        pl.kernel,
        out_type=jax.ShapeDtypeStruct((VOCAB, 2 * DIM), jnp.float32),
        mesh=mesh,
        compiler_params=params,
        scratch_types=[
            pltpu.VMEM((DIM, BLK), jnp.float32),
            pltpu.VMEM((DIM, BLK), jnp.float32),
            pltpu.VMEM((BLK, 2 * DIM), jnp.float32),
            pltpu.VMEM((BLK, 2 * DIM), jnp.float32),
            pltpu.VMEM((DIM, TAILN), jnp.float32),
            pltpu.VMEM((TAILN, 2 * DIM), jnp.float32),
            pltpu.SemaphoreType.DMA,
            pltpu.SemaphoreType.DMA,
            pltpu.SemaphoreType.DMA,
            pltpu.SemaphoreType.DMA,
        ],
    )(_tr_body)

    pool = functools.partial(
        pl.kernel,
        out_type=jax.ShapeDtypeStruct((BATCH, 2 * DIM), jnp.float32),
        mesh=mesh,
        compiler_params=params,
        scratch_types=[
            pltpu.VMEM((XCH, BPW), jnp.int32),
            pltpu.VMEM((BPW, SEQ), jnp.int32),
            pltpu.VMEM((SEQ, 2 * DIM), jnp.float32),
            pltpu.VMEM((SEQ, 2 * DIM), jnp.float32),
            pltpu.VMEM((BPW, 2 * DIM), jnp.float32),
            pltpu.SemaphoreType.DMA,
            pltpu.SemaphoreType.DMA,
        ],
    )(_pool_body)

    padded = transpose(tt)
    out128 = pool(x, padded)
    return out128[:, :DIM]
